# Initial kernel scaffold; baseline (speedup 1.0000x reference)
import jax
import jax.numpy as jnp
from jax.experimental import pallas as pl


def kernel(x, lengths, emb, W1, b1, W2, b2):
    raise NotImplementedError("write your pallas kernel here")


# speedup vs baseline: 9.4299x; 9.4299x over previous
import functools

import jax
import jax.numpy as jnp
from jax import lax
from jax.experimental import pallas as pl
from jax.experimental.pallas import tpu as pltpu
from jax.experimental.pallas import tpu_sc as plsc

_VOCAB = 100000
_D = 64
_HID = 256
_NCLS = 128
_B = 4096
_L = 200

_NC = 2
_NS = 16
_NW = _NC * _NS
_BPT = _B // _NW
_LC = 100
_NCHUNK = _L // _LC
_NV = _D // 16

_ROWS_PER_IT = 4


def _sc_pool_body(x_hbm, emb_hbm, sums_hbm, idx_v, rows_v, out_v, sem):
    wid = lax.axis_index("s") * _NC + lax.axis_index("c")
    base = wid * _BPT
    pltpu.sync_copy(x_hbm.at[pl.ds(base, _BPT)], idx_v)

    def bag_body(b, carry):
        cps = [
            pltpu.async_copy(
                emb_hbm.at[idx_v.at[b, c]],
                rows_v.at[pl.ds(c * _LC, _LC)],
                sem,
            )
            for c in range(_NCHUNK)
        ]
        for cp in cps:
            cp.wait()

        def row_body(r, accs):
            r0 = r * _ROWS_PER_IT
            new = list(accs)
            for dr in range(_ROWS_PER_IT):
                for cc in range(_NV):
                    new[cc] = new[cc] + rows_v[r0 + dr, pl.ds(cc * 16, 16)]
            return tuple(new)

        accs = lax.fori_loop(
            0, _L // _ROWS_PER_IT, row_body,
            tuple(jnp.zeros((16,), jnp.float32) for _ in range(_NV)),
        )
        for cc in range(_NV):
            out_v[b, pl.ds(cc * 16, 16)] = accs[cc]
        return carry

    lax.fori_loop(0, _BPT, bag_body, 0)
    pltpu.sync_copy(out_v, sums_hbm.at[pl.ds(base, _BPT)])


@functools.cache
def _sc_pool():
    return pl.kernel(
        _sc_pool_body,
        out_type=jax.ShapeDtypeStruct((_B, _D), jnp.float32),
        mesh=plsc.VectorSubcoreMesh(core_axis_name="c", subcore_axis_name="s"),
        scratch_types=[
            pltpu.VMEM((_BPT, _NCHUNK, _LC), jnp.int32),
            pltpu.VMEM((_L, _D), jnp.float32),
            pltpu.VMEM((_BPT, _D), jnp.float32),
            pltpu.SemaphoreType.DMA,
        ],
        compiler_params=pltpu.CompilerParams(use_tc_tiling_on_sc=False),
    )


_BM = 512


def _mlp_body(sums_ref, x_ref, w1_ref, b1_ref, w2_ref, b2_ref, out_ref):
    cnt = jnp.sum((x_ref[...] != 0).astype(jnp.float32), axis=1, keepdims=True)
    mean = sums_ref[...] / jnp.maximum(cnt, 1.0)
    h = jnp.maximum(
        jnp.dot(mean, w1_ref[...], preferred_element_type=jnp.float32)
        + b1_ref[...],
        0.0,
    )
    out_ref[...] = (
        jnp.dot(h, w2_ref[...], preferred_element_type=jnp.float32)
        + b2_ref[...]
    )


def _mlp(sums, x, W1, b1, W2, b2):
    return pl.pallas_call(
        _mlp_body,
        grid=(_B // _BM,),
        in_specs=[
            pl.BlockSpec((_BM, _D), lambda i: (i, 0)),
            pl.BlockSpec((_BM, _L), lambda i: (i, 0)),
            pl.BlockSpec((_D, _HID), lambda i: (0, 0)),
            pl.BlockSpec((1, _HID), lambda i: (0, 0)),
            pl.BlockSpec((_HID, _NCLS), lambda i: (0, 0)),
            pl.BlockSpec((1, _NCLS), lambda i: (0, 0)),
        ],
        out_specs=pl.BlockSpec((_BM, _NCLS), lambda i: (i, 0)),
        out_shape=jax.ShapeDtypeStruct((_B, _NCLS), jnp.float32),
    )(sums, x, W1, b1, W2, b2)


def kernel(x, lengths, emb, W1, b1, W2, b2):
    del lengths
    x = x.astype(jnp.int32)
    sums = _sc_pool()(x.reshape(_B, _NCHUNK, _LC), emb)
    return _mlp(sums, x, W1, b1.reshape(1, _HID), W2, b2.reshape(1, _NCLS))


# speedup vs baseline: 13.6622x; 1.4488x over previous
import functools

import jax
import jax.numpy as jnp
from jax import lax
from jax.experimental import pallas as pl
from jax.experimental.pallas import tpu as pltpu
from jax.experimental.pallas import tpu_sc as plsc

_VOCAB = 100000
_D = 64
_HID = 256
_NCLS = 128
_B = 4096
_L = 200

_NC = 2
_NS = 16
_NW = _NC * _NS
_BPT = _B // _NW
_LC = 100
_NCHUNK = _L // _LC
_NV = _D // 16

_ROWS_PER_IT = 4


def _sc_pool_body(x_hbm, emb_hbm, sums_hbm, idx_v, rows_v, out_v, sem0, sem1):
    wid = lax.axis_index("s") * _NC + lax.axis_index("c")
    base = wid * _BPT
    pltpu.sync_copy(x_hbm.at[pl.ds(base, _BPT)], idx_v)
    sems = (sem0, sem1)

    def fire(bag, buf):
        return [
            pltpu.async_copy(
                emb_hbm.at[idx_v.at[bag, c]],
                rows_v.at[buf, pl.ds(c * _LC, _LC)],
                sems[buf],
            )
            for c in range(_NCHUNK)
        ]

    def drain(buf):
        for c in range(_NCHUNK):
            pltpu.make_async_copy(
                emb_hbm.at[idx_v.at[0, c]],
                rows_v.at[buf, pl.ds(c * _LC, _LC)],
                sems[buf],
            ).wait()

    def accumulate(bag, buf):
        def row_body(r, accs):
            r0 = r * _ROWS_PER_IT
            new = list(accs)
            for dr in range(_ROWS_PER_IT):
                for cc in range(_NV):
                    new[cc] = new[cc] + rows_v[buf, r0 + dr, pl.ds(cc * 16, 16)]
            return tuple(new)

        accs = lax.fori_loop(
            0, _L // _ROWS_PER_IT, row_body,
            tuple(jnp.zeros((16,), jnp.float32) for _ in range(_NV)),
        )
        for cc in range(_NV):
            out_v[bag, pl.ds(cc * 16, 16)] = accs[cc]

    fire(0, 0)

    def pair_body(i, carry):
        bag = 2 * i
        fire(jnp.minimum(bag + 1, _BPT - 1), 1)
        drain(0)
        accumulate(bag, 0)
        fire(jnp.minimum(bag + 2, _BPT - 1), 0)
        drain(1)
        accumulate(bag + 1, 1)
        return carry

    lax.fori_loop(0, _BPT // 2, pair_body, 0)
    drain(0)
    pltpu.sync_copy(out_v, sums_hbm.at[pl.ds(base, _BPT)])


@functools.cache
def _sc_pool():
    return pl.kernel(
        _sc_pool_body,
        out_type=jax.ShapeDtypeStruct((_B, _D), jnp.float32),
        mesh=plsc.VectorSubcoreMesh(core_axis_name="c", subcore_axis_name="s"),
        scratch_types=[
            pltpu.VMEM((_BPT, _NCHUNK, _LC), jnp.int32),
            pltpu.VMEM((2, _L, _D), jnp.float32),
            pltpu.VMEM((_BPT, _D), jnp.float32),
            pltpu.SemaphoreType.DMA,
            pltpu.SemaphoreType.DMA,
        ],
        compiler_params=pltpu.CompilerParams(use_tc_tiling_on_sc=False),
    )


_BM = 512


def _mlp_body(sums_ref, x_ref, w1_ref, b1_ref, w2_ref, b2_ref, out_ref):
    cnt = jnp.sum((x_ref[...] != 0).astype(jnp.float32), axis=1, keepdims=True)
    mean = sums_ref[...] / jnp.maximum(cnt, 1.0)
    h = jnp.maximum(
        jnp.dot(mean, w1_ref[...], preferred_element_type=jnp.float32)
        + b1_ref[...],
        0.0,
    )
    out_ref[...] = (
        jnp.dot(h, w2_ref[...], preferred_element_type=jnp.float32)
        + b2_ref[...]
    )


def _mlp(sums, x, W1, b1, W2, b2):
    return pl.pallas_call(
        _mlp_body,
        grid=(_B // _BM,),
        in_specs=[
            pl.BlockSpec((_BM, _D), lambda i: (i, 0)),
            pl.BlockSpec((_BM, _L), lambda i: (i, 0)),
            pl.BlockSpec((_D, _HID), lambda i: (0, 0)),
            pl.BlockSpec((1, _HID), lambda i: (0, 0)),
            pl.BlockSpec((_HID, _NCLS), lambda i: (0, 0)),
            pl.BlockSpec((1, _NCLS), lambda i: (0, 0)),
        ],
        out_specs=pl.BlockSpec((_BM, _NCLS), lambda i: (i, 0)),
        out_shape=jax.ShapeDtypeStruct((_B, _NCLS), jnp.float32),
    )(sums, x, W1, b1, W2, b2)


def kernel(x, lengths, emb, W1, b1, W2, b2):
    del lengths
    x = x.astype(jnp.int32)
    sums = _sc_pool()(x.reshape(_B, _NCHUNK, _LC), emb)
    return _mlp(sums, x, W1, b1.reshape(1, _HID), W2, b2.reshape(1, _NCLS))
